# Initial kernel scaffold; baseline (speedup 1.0000x reference)
import jax
import jax.numpy as jnp
from jax.experimental import pallas as pl


def kernel(ray_origins, ray_directions, density, sh_coeffs, num_samples):
    raise NotImplementedError("write your pallas kernel here")


# speedup vs baseline: 1.8147x; 1.8147x over previous
import functools

import jax
import jax.numpy as jnp
from jax import lax
from jax.experimental import pallas as pl
from jax.experimental.pallas import tpu as pltpu
from jax.experimental.pallas import tpu_sc as plsc

RES = 128
V = RES * RES * RES
NUM_COEFFS = 9
S = 64
B = 8192
NEAR, FAR = 0.1, 10.0
NC, NS = 2, 16
NW = NC * NS
RPW = B // NW
ROWW = 32
GW = 16
NG = S // GW
PR = 16

_f32 = jnp.float32
_i32 = jnp.int32


def _bcast_i(x):
    return jnp.full((GW,), x, _i32)


def _bcast_f(x):
    return jnp.full((GW,), x, _f32)


def _lane(vec, i):
    return jnp.take_along_axis(vec, _bcast_i(i), axis=0)


def _sc_render(table_hbm, perray_hbm, dn_hbm, tpack_hbm, out_hbm,
               perray_v, dn_v, tpack_v, idx_v, w_v, rows_v, sigw, colw,
               rgb_v, sem):
    wid = lax.axis_index("s") * NC + lax.axis_index("c")
    pltpu.sync_copy(perray_hbm.at[pl.ds(wid * RPW * PR, RPW * PR)], perray_v)
    pltpu.sync_copy(dn_hbm.at[pl.ds(wid * RPW, RPW)], dn_v)
    pltpu.sync_copy(tpack_hbm, tpack_v)

    iota = lax.iota(_i32, GW)
    iota_s = iota * S

    def phase_a(rr, bn):
        row = perray_v[pl.ds(rr * PR, PR)]
        ox, oy, oz = _lane(row, 10), _lane(row, 11), _lane(row, 12)
        dx, dy, dz = _lane(row, 13), _lane(row, 14), _lane(row, 15)
        boff = bn * (8 * S)
        for g in range(NG):
            t = tpack_v[pl.ds(g * GW, GW)]

            def voxify(o, d):
                v = (o + t * d + 1.0) * (RES / 2.0)
                v = jnp.minimum(jnp.maximum(v, 0.0), RES - 1.0)
                ci = v.astype(_i32)
                f = v - ci.astype(_f32)
                c1 = jnp.minimum(ci + 1, RES - 1)
                return ci, c1, f

            x0, x1, fx = voxify(ox, dx)
            y0, y1, fy = voxify(oy, dy)
            z0, z1, fz = voxify(oz, dz)
            a0 = x0 * (RES * RES)
            a1 = x1 * (RES * RES)
            b0 = y0 * RES
            b1 = y1 * RES
            ab = (a0 + b0, a0 + b1, a1 + b0, a1 + b1)
            gx, gy, gz = 1.0 - fx, 1.0 - fy, 1.0 - fz
            wx = (gx * gy, gx * fy, fx * gy, fx * fy)
            for c in range(8):
                xy = (c >> 1)
                zc = z0 if (c & 1) == 0 else z1
                wz = gz if (c & 1) == 0 else fz
                idx_v[pl.ds(boff + c * S + g * GW, GW)] = ab[xy] + zc
                w_v[pl.ds(boff + c * S + g * GW, GW)] = wx[xy] * wz
        for c in range(8):
            pltpu.async_copy(
                table_hbm.at[idx_v.at[pl.ds(boff + c * S, S)]],
                rows_v.at[pl.ds((bn * 8 + c) * S, S), :],
                sem)

    phase_a(jnp.int32(0), jnp.int32(0))

    def body(r, _):
        bc = lax.rem(r, 2)
        bn = lax.rem(r + 1, 2)
        bcoff = bc * (8 * S)
        for c in range(8):
            pltpu.make_async_copy(
                table_hbm.at[idx_v.at[pl.ds(bcoff + c * S, S)]],
                rows_v.at[pl.ds((bc * 8 + c) * S, S), :],
                sem).wait()

        @pl.when(r + 1 < RPW)
        def _():
            phase_a(r + 1, bn)

        brow = perray_v[pl.ds(r * PR, PR)]
        bas = [_lane(brow, k) for k in range(NUM_COEFFS)]
        rloc = lax.rem(r, GW)
        for g in range(NG):
            rvs = [(bc * 8 + c) * S + g * GW + iota for c in range(8)]
            wvs = [w_v[pl.ds(bcoff + c * S + g * GW, GW)] for c in range(8)]
            colacc = [None, None, None]
            sigma = None
            for j in range(28):
                jv = _bcast_i(j)
                acc = wvs[0] * plsc.load_gather(rows_v, [rvs[0], jv])
                for c in range(1, 8):
                    acc = acc + wvs[c] * plsc.load_gather(rows_v, [rvs[c], jv])
                if j < 27:
                    ch, k = j // NUM_COEFFS, j % NUM_COEFFS
                    colacc[ch] = acc * bas[k] if colacc[ch] is None \
                        else colacc[ch] + acc * bas[k]
                else:
                    sigma = acc
            sigw[pl.ds(rloc * S + g * GW, GW)] = sigma
            for ch in range(3):
                col = 1.0 / (1.0 + jnp.exp(-colacc[ch]))
                colw[pl.ds(ch * (GW * S) + rloc * S + g * GW, GW)] = col

        @pl.when(rloc == GW - 1)
        def _():
            r0 = r - (GW - 1)
            dvec = dn_v[pl.ds(r0, GW)]
            trans = _bcast_f(1.0)
            rgb = [_bcast_f(0.0) for _ in range(3)]
            for sb in range(NG):
                dch = tpack_v[pl.ds(S + sb * GW, GW)]
                for sl in range(GW):
                    s = sb * GW + sl
                    sg = plsc.load_gather(sigw, [iota_s + s])
                    dist = _lane(dch, sl) * dvec
                    a = 1.0 - jnp.exp(-jnp.maximum(sg, 0.0) * dist)
                    wgt = a * trans
                    for ch in range(3):
                        cv = plsc.load_gather(colw, [iota_s + (ch * (GW * S) + s)])
                        rgb[ch] = rgb[ch] + wgt * cv
                    trans = trans * (1.0 - a + 1e-10)
            for ch in range(3):
                rgb_v[pl.ds(ch * RPW + r0, GW)] = rgb[ch]

        return 0

    lax.fori_loop(0, RPW, body, 0)
    for ch in range(3):
        pltpu.sync_copy(rgb_v.at[pl.ds(ch * RPW, RPW)],
                        out_hbm.at[pl.ds(ch * B + wid * RPW, RPW)])


_render = functools.partial(
    pl.kernel,
    out_type=jax.ShapeDtypeStruct((3 * B,), _f32),
    mesh=plsc.VectorSubcoreMesh(core_axis_name="c", subcore_axis_name="s"),
    compiler_params=pltpu.CompilerParams(use_tc_tiling_on_sc=False,
                                         needs_layout_passes=False),
    scratch_types=[
        pltpu.VMEM((RPW * PR,), _f32),
        pltpu.VMEM((RPW,), _f32),
        pltpu.VMEM((2 * S,), _f32),
        pltpu.VMEM((2 * 8 * S,), _i32),
        pltpu.VMEM((2 * 8 * S,), _f32),
        pltpu.VMEM((16 * S, ROWW), _f32),
        pltpu.VMEM((GW * S,), _f32),
        pltpu.VMEM((3 * GW * S,), _f32),
        pltpu.VMEM((3 * RPW,), _f32),
        pltpu.SemaphoreType.DMA,
    ],
)(_sc_render)


def kernel(ray_origins, ray_directions, density, sh_coeffs, num_samples):
    del num_samples
    sh_flat = sh_coeffs.reshape(V, 3 * NUM_COEFFS)
    table = jnp.concatenate(
        [sh_flat, density.reshape(V, 1), jnp.zeros((V, 4), _f32)], axis=1)

    dnorm = jnp.sqrt(jnp.sum(ray_directions * ray_directions, axis=-1,
                             keepdims=True))
    d = ray_directions / (dnorm + 1e-8)
    x, y, z = d[:, 0], d[:, 1], d[:, 2]
    xx, yy, zz = x * x, y * y, z * z
    basis = jnp.stack([
        jnp.full((B,), 0.28209479177387814, _f32),
        -0.48860251190291987 * y,
        0.48860251190291987 * z,
        -0.48860251190291987 * x,
        1.0925484305920792 * x * y,
        -1.0925484305920792 * y * z,
        0.31539156525252005 * (2.0 * zz - xx - yy),
        -1.0925484305920792 * x * z,
        0.5462742152960396 * (xx - yy),
    ], axis=-1)
    pad = jnp.zeros((B, PR - NUM_COEFFS - 6), _f32)
    perray = jnp.concatenate(
        [basis, pad, ray_origins, ray_directions], axis=1).reshape(-1)

    t_vals = jnp.linspace(NEAR, FAR, S, dtype=_f32)
    dists = jnp.concatenate([t_vals[1:] - t_vals[:-1],
                             jnp.full((1,), 1e10, _f32)])
    tpack = jnp.concatenate([t_vals, dists])

    out = _render(table, perray, dnorm.reshape(-1), tpack)
    return out.reshape(3, B).T
